# Initial kernel scaffold; baseline (speedup 1.0000x reference)
import functools

import jax
import jax.numpy as jnp
from jax import lax
from jax.experimental import pallas as pl
from jax.experimental.pallas import tpu as pltpu
from jax.experimental.pallas import tpu_sc as plsc

N = 10000
D = 128
E = 320000

NC = 2
NS = 16
NW = NC * NS

CHUNK = 128
N_PAD = 10016
Z = N_PAD // NS
EDGES_PER_TILE = 10240
E_PAD = EDGES_PER_TILE * NW
K_CHUNKS = EDGES_PER_TILE // CHUNK
CW = 16

_mesh = plsc.VectorSubcoreMesh(core_axis_name="c", subcore_axis_name="s")


def _agg_body(with_counts, *refs):
    if with_counts:
        (y_hbm, src_hbm, dst_hbm, ones_hbm, zrow_hbm, zcnt_hbm,
         out_hbm, cnt_hbm, srcv, dstv, rows, ones_v, acc, cacc, sem) = refs
    else:
        (y_hbm, src_hbm, dst_hbm, zrow_hbm,
         out_hbm, srcv, dstv, rows, acc, sem) = refs

    cid = lax.axis_index("c")
    sid = lax.axis_index("s")
    wid = sid * NC + cid
    base = wid * EDGES_PER_TILE
    zoff = sid * Z

    pltpu.sync_copy(zrow_hbm.at[pl.ds(zoff, Z)], acc.at[pl.ds(zoff, Z)])
    if with_counts:
        pltpu.sync_copy(zcnt_hbm.at[pl.ds(zoff, Z)], cacc.at[pl.ds(zoff, Z)])
        pltpu.sync_copy(ones_hbm, ones_v)
    plsc.subcore_barrier()

    def chunk(j, carry):
        off = base + j * CHUNK
        pltpu.sync_copy(src_hbm.at[pl.ds(off, CHUNK)], srcv)
        pltpu.sync_copy(dst_hbm.at[pl.ds(off, CHUNK)], dstv)
        pltpu.async_copy(y_hbm.at[srcv], rows, sem).wait()
        pltpu.sync_copy(rows, acc.at[dstv], add=True)
        if with_counts:
            pltpu.sync_copy(ones_v, cacc.at[dstv], add=True)
        return carry

    lax.fori_loop(0, K_CHUNKS, chunk, 0)
    plsc.subcore_barrier()

    orow = cid * N_PAD + zoff
    pltpu.sync_copy(acc.at[pl.ds(zoff, Z)], out_hbm.at[pl.ds(orow, Z)])
    if with_counts:
        pltpu.sync_copy(cacc.at[pl.ds(zoff, Z)], cnt_hbm.at[pl.ds(orow, Z)])


_agg_with_counts = functools.partial(
    pl.kernel,
    out_type=(jax.ShapeDtypeStruct((NC * N_PAD, D), jnp.float32),
              jax.ShapeDtypeStruct((NC * N_PAD, CW), jnp.float32)),
    mesh=_mesh,
    scratch_types=[
        pltpu.VMEM((CHUNK,), jnp.int32),
        pltpu.VMEM((CHUNK,), jnp.int32),
        pltpu.VMEM((CHUNK, D), jnp.float32),
        pltpu.VMEM((CHUNK, CW), jnp.float32),
        pltpu.VMEM_SHARED((N_PAD, D), jnp.float32),
        pltpu.VMEM_SHARED((N_PAD, CW), jnp.float32),
        pltpu.SemaphoreType.DMA,
    ],
)(functools.partial(_agg_body, True))

_agg_no_counts = functools.partial(
    pl.kernel,
    out_type=jax.ShapeDtypeStruct((NC * N_PAD, D), jnp.float32),
    mesh=_mesh,
    scratch_types=[
        pltpu.VMEM((CHUNK,), jnp.int32),
        pltpu.VMEM((CHUNK,), jnp.int32),
        pltpu.VMEM((CHUNK, D), jnp.float32),
        pltpu.VMEM_SHARED((N_PAD, D), jnp.float32),
        pltpu.SemaphoreType.DMA,
    ],
)(functools.partial(_agg_body, False))


def _mm_t(a, w):
    return lax.dot_general(a, w, (((1,), (1,)), ((), ())),
                           preferred_element_type=jnp.float32)


def _tc_pre_body(x_ref, wl_ref, wr_ref, b_ref, yl_ref, yr_ref):
    xv = x_ref[...]
    yl_ref[...] = _mm_t(xv, wl_ref[...])
    yr_ref[...] = _mm_t(xv, wr_ref[...]) + b_ref[...]


def _tc_mid_body(p_ref, c_ref, yr_ref, wl_ref, wr_ref, b_ref,
                 yl2_ref, yr2_ref):
    s = p_ref[0] + p_ref[1]
    cnt = jnp.maximum(c_ref[0, :, :1] + c_ref[1, :, :1], 1.0)
    h = jnp.maximum(s / cnt + yr_ref[...], 0.0)
    yl2_ref[...] = _mm_t(h, wl_ref[...])
    yr2_ref[...] = _mm_t(h, wr_ref[...]) + b_ref[...]


def _tc_post_body(p_ref, c_ref, yr_ref, o_ref):
    s = p_ref[0] + p_ref[1]
    cnt = jnp.maximum(c_ref[0, :, :1] + c_ref[1, :, :1], 1.0)
    o_ref[...] = s / cnt + yr_ref[...]


_tc_pre = pl.pallas_call(
    _tc_pre_body,
    out_shape=(jax.ShapeDtypeStruct((N, D), jnp.float32),
               jax.ShapeDtypeStruct((N, D), jnp.float32)),
)

_tc_mid = pl.pallas_call(
    _tc_mid_body,
    out_shape=(jax.ShapeDtypeStruct((N, D), jnp.float32),
               jax.ShapeDtypeStruct((N, D), jnp.float32)),
)

_tc_post = pl.pallas_call(
    _tc_post_body,
    out_shape=jax.ShapeDtypeStruct((N, D), jnp.float32),
)


def kernel(x, edge_index, W1_l, W1_r, b1, W2_l, W2_r, b2):
    src = edge_index[0].astype(jnp.int32)
    dst = edge_index[1].astype(jnp.int32)
    pad = E_PAD - E
    src = jnp.concatenate([src, jnp.zeros((pad,), jnp.int32)])
    dst = jnp.concatenate([dst, jnp.full((pad,), N, jnp.int32)])

    ones_blk = jnp.ones((CHUNK, CW), jnp.float32)
    zrow = jnp.zeros((N_PAD, D), jnp.float32)
    zcnt = jnp.zeros((N_PAD, CW), jnp.float32)

    b1r = b1.reshape(1, D)
    b2r = b2.reshape(1, D)

    y1l, y1r = _tc_pre(x, W1_l, W1_r, b1r)
    p1, cnts = _agg_with_counts(y1l, src, dst, ones_blk, zrow, zcnt)

    p1 = p1.reshape(NC, N_PAD, D)[:, :N]
    cnt = cnts.reshape(NC, N_PAD, CW)[:, :N]

    y2l, y2r = _tc_mid(p1, cnt, y1r, W2_l, W2_r, b2r)
    p2 = _agg_no_counts(y2l, src, dst, zrow)
    p2 = p2.reshape(NC, N_PAD, D)[:, :N]

    return _tc_post(p2, cnt, y2r)


# speedup vs baseline: 2.7395x; 2.7395x over previous
import functools

import jax
import jax.numpy as jnp
from jax import lax
from jax.experimental import pallas as pl
from jax.experimental.pallas import tpu as pltpu
from jax.experimental.pallas import tpu_sc as plsc

N = 10000
D = 128
E = 320000

NC = 2
NS = 16
NW = NC * NS

CHUNK = 128
N_PAD = 10240
Z = N_PAD // NS
N_PIECES = Z // CHUNK
EDGES_PER_TILE = 10240
E_PAD = EDGES_PER_TILE * NW
K_CHUNKS = EDGES_PER_TILE // CHUNK

_mesh = plsc.VectorSubcoreMesh(core_axis_name="c", subcore_axis_name="s")


def _fill_rows(ref, value):
    vec = jnp.full((16,), value, jnp.float32)

    def body(i, carry):
        for j in range(D // 16):
            ref[i, pl.ds(j * 16, 16)] = vec
        return carry

    lax.fori_loop(0, CHUNK, body, 0)


def _zero_acc(rows, acc, zoff):
    _fill_rows(rows, 0.0)
    for p in range(N_PIECES):
        pltpu.sync_copy(rows, acc.at[pl.ds(zoff + p * CHUNK, CHUNK)])


def _flush_acc(acc, rows, out_hbm, zoff, orow):
    for p in range(N_PIECES):
        pltpu.sync_copy(acc.at[pl.ds(zoff + p * CHUNK, CHUNK)], rows)
        pltpu.sync_copy(rows, out_hbm.at[pl.ds(orow + p * CHUNK, CHUNK)])


def _agg_body(y_hbm, src_hbm, dst_hbm, out_hbm, srcv, dstv, rows, acc, sem):
    cid = lax.axis_index("c")
    sid = lax.axis_index("s")
    wid = sid * NC + cid
    base = wid * EDGES_PER_TILE
    zoff = sid * Z

    _zero_acc(rows, acc, zoff)
    plsc.subcore_barrier()

    def chunk(j, carry):
        off = base + j * CHUNK
        pltpu.sync_copy(src_hbm.at[pl.ds(off, CHUNK)], srcv)
        pltpu.sync_copy(dst_hbm.at[pl.ds(off, CHUNK)], dstv)
        pltpu.async_copy(y_hbm.at[srcv], rows, sem).wait()
        pltpu.sync_copy(rows, acc.at[dstv], add=True)
        return carry

    lax.fori_loop(0, K_CHUNKS, chunk, 0)
    plsc.subcore_barrier()
    _flush_acc(acc, rows, out_hbm, zoff, cid * N_PAD + zoff)


def _cnt_body(dst_hbm, out_hbm, dstv, rows, acc, sem):
    cid = lax.axis_index("c")
    sid = lax.axis_index("s")
    wid = sid * NC + cid
    base = wid * EDGES_PER_TILE
    zoff = sid * Z

    _zero_acc(rows, acc, zoff)
    _fill_rows(rows, 1.0)
    plsc.subcore_barrier()

    def chunk(j, carry):
        off = base + j * CHUNK
        pltpu.sync_copy(dst_hbm.at[pl.ds(off, CHUNK)], dstv)
        pltpu.sync_copy(rows, acc.at[dstv], add=True)
        return carry

    lax.fori_loop(0, K_CHUNKS, chunk, 0)
    plsc.subcore_barrier()
    _flush_acc(acc, rows, out_hbm, zoff, cid * N_PAD + zoff)


_agg = functools.partial(
    pl.kernel,
    out_type=jax.ShapeDtypeStruct((NC * N_PAD, D), jnp.float32),
    mesh=_mesh,
    scratch_types=[
        pltpu.VMEM((CHUNK,), jnp.int32),
        pltpu.VMEM((CHUNK,), jnp.int32),
        pltpu.VMEM((CHUNK, D), jnp.float32),
        pltpu.VMEM_SHARED((N_PAD, D), jnp.float32),
        pltpu.SemaphoreType.DMA,
    ],
)(_agg_body)

_cnt = functools.partial(
    pl.kernel,
    out_type=jax.ShapeDtypeStruct((NC * N_PAD, D), jnp.float32),
    mesh=_mesh,
    scratch_types=[
        pltpu.VMEM((CHUNK,), jnp.int32),
        pltpu.VMEM((CHUNK, D), jnp.float32),
        pltpu.VMEM_SHARED((N_PAD, D), jnp.float32),
        pltpu.SemaphoreType.DMA,
    ],
)(_cnt_body)


def _mm_t(a, w):
    return lax.dot_general(a, w, (((1,), (1,)), ((), ())),
                           preferred_element_type=jnp.float32)


def _tc_pre_body(x_ref, wl_ref, wr_ref, b_ref, yl_ref, yr_ref):
    xv = x_ref[...]
    yl_ref[...] = _mm_t(xv, wl_ref[...])
    yr_ref[...] = _mm_t(xv, wr_ref[...]) + b_ref[...]


def _tc_mid_body(p_ref, c_ref, yr_ref, wl_ref, wr_ref, b_ref,
                 yl2_ref, yr2_ref):
    s = p_ref[0] + p_ref[1]
    cnt = jnp.maximum(c_ref[0, :, :1] + c_ref[1, :, :1], 1.0)
    h = jnp.maximum(s / cnt + yr_ref[...], 0.0)
    yl2_ref[...] = _mm_t(h, wl_ref[...])
    yr2_ref[...] = _mm_t(h, wr_ref[...]) + b_ref[...]


def _tc_post_body(p_ref, c_ref, yr_ref, o_ref):
    s = p_ref[0] + p_ref[1]
    cnt = jnp.maximum(c_ref[0, :, :1] + c_ref[1, :, :1], 1.0)
    o_ref[...] = s / cnt + yr_ref[...]


_tc_pre = pl.pallas_call(
    _tc_pre_body,
    out_shape=(jax.ShapeDtypeStruct((N, D), jnp.float32),
               jax.ShapeDtypeStruct((N, D), jnp.float32)),
)

_tc_mid = pl.pallas_call(
    _tc_mid_body,
    out_shape=(jax.ShapeDtypeStruct((N, D), jnp.float32),
               jax.ShapeDtypeStruct((N, D), jnp.float32)),
)

_tc_post = pl.pallas_call(
    _tc_post_body,
    out_shape=jax.ShapeDtypeStruct((N, D), jnp.float32),
)


def kernel(x, edge_index, W1_l, W1_r, b1, W2_l, W2_r, b2):
    src = edge_index[0].astype(jnp.int32)
    dst = edge_index[1].astype(jnp.int32)
    pad = E_PAD - E
    src = jnp.concatenate([src, jnp.zeros((pad,), jnp.int32)])
    dst = jnp.concatenate([dst, jnp.full((pad,), N, jnp.int32)])

    b1r = b1.reshape(1, D)
    b2r = b2.reshape(1, D)

    y1l, y1r = _tc_pre(x, W1_l, W1_r, b1r)
    cnts = _cnt(dst)
    p1 = _agg(y1l, src, dst)

    p1 = p1.reshape(NC, N_PAD, D)[:, :N]
    cnt = cnts.reshape(NC, N_PAD, D)[:, :N]

    y2l, y2r = _tc_mid(p1, cnt, y1r, W2_l, W2_r, b2r)
    p2 = _agg(y2l, src, dst)
    p2 = p2.reshape(NC, N_PAD, D)[:, :N]

    return _tc_post(p2, cnt, y2r)


# speedup vs baseline: 3.4922x; 1.2747x over previous
import functools

import jax
import jax.numpy as jnp
from jax import lax
from jax.experimental import pallas as pl
from jax.experimental.pallas import tpu as pltpu
from jax.experimental.pallas import tpu_sc as plsc

N = 10000
D = 128
E = 320000

NC = 2
NS = 16
NW = NC * NS

CHUNK = 128
N_PAD = 10240
Z = N_PAD // NS
N_PIECES = Z // CHUNK
EDGES_PER_TILE = 10240
E_PAD = EDGES_PER_TILE * NW
K_CHUNKS = EDGES_PER_TILE // CHUNK

_mesh = plsc.VectorSubcoreMesh(core_axis_name="c", subcore_axis_name="s")


def _fill_rows(ref, value):
    vec = jnp.full((16,), value, jnp.float32)

    def body(i, carry):
        for j in range(D // 16):
            ref[i, pl.ds(j * 16, 16)] = vec
        return carry

    lax.fori_loop(0, CHUNK, body, 0)


def _zero_acc(rows, acc, zoff):
    _fill_rows(rows, 0.0)
    for p in range(N_PIECES):
        pltpu.sync_copy(rows, acc.at[pl.ds(zoff + p * CHUNK, CHUNK)])


def _flush_acc(acc, rows, out_hbm, zoff, orow):
    for p in range(N_PIECES):
        pltpu.sync_copy(acc.at[pl.ds(zoff + p * CHUNK, CHUNK)], rows)
        pltpu.sync_copy(rows, out_hbm.at[pl.ds(orow + p * CHUNK, CHUNK)])


def _agg_body(y_hbm, edges_hbm, out_hbm, ib0, ib1, rows0, rows1,
              acc, sem0, sem1):
    cid = lax.axis_index("c")
    sid = lax.axis_index("s")
    wid = sid * NC + cid
    bc = wid * K_CHUNKS
    zoff = sid * Z

    _zero_acc(rows0, acc, zoff)
    plsc.subcore_barrier()

    def load_idx(c, ib):
        pltpu.sync_copy(edges_hbm.at[bc + c], ib)

    def start_gather(ib, rows, sem):
        pltpu.async_copy(y_hbm.at[ib.at[0]], rows, sem)

    def wait_gather(ib, rows, sem):
        pltpu.make_async_copy(y_hbm.at[ib.at[0]], rows, sem).wait()

    def scatter(rows, ib):
        pltpu.sync_copy(rows, acc.at[ib.at[1]], add=True)

    load_idx(0, ib0)
    start_gather(ib0, rows0, sem0)
    G = K_CHUNKS // 2

    def pair(g, carry):
        c = 2 * g
        load_idx(c + 1, ib1)
        start_gather(ib1, rows1, sem1)
        wait_gather(ib0, rows0, sem0)
        scatter(rows0, ib0)

        @pl.when(g + 1 < G)
        def _():
            load_idx(c + 2, ib0)
            start_gather(ib0, rows0, sem0)

        wait_gather(ib1, rows1, sem1)
        scatter(rows1, ib1)
        return carry

    lax.fori_loop(0, G, pair, 0)
    plsc.subcore_barrier()
    _flush_acc(acc, rows0, out_hbm, zoff, cid * N_PAD + zoff)


def _cnt_body(dst_hbm, out_hbm, dstv, rows, acc, sem):
    cid = lax.axis_index("c")
    sid = lax.axis_index("s")
    wid = sid * NC + cid
    base = wid * EDGES_PER_TILE
    zoff = sid * Z

    _zero_acc(rows, acc, zoff)
    _fill_rows(rows, 1.0)
    plsc.subcore_barrier()

    def chunk(j, carry):
        off = base + j * CHUNK
        pltpu.sync_copy(dst_hbm.at[pl.ds(off, CHUNK)], dstv)
        pltpu.sync_copy(rows, acc.at[dstv], add=True)
        return carry

    lax.fori_loop(0, K_CHUNKS, chunk, 0)
    plsc.subcore_barrier()
    _flush_acc(acc, rows, out_hbm, zoff, cid * N_PAD + zoff)


_agg = functools.partial(
    pl.kernel,
    out_type=jax.ShapeDtypeStruct((NC * N_PAD, D), jnp.float32),
    mesh=_mesh,
    scratch_types=[
        pltpu.VMEM((2, CHUNK), jnp.int32),
        pltpu.VMEM((2, CHUNK), jnp.int32),
        pltpu.VMEM((CHUNK, D), jnp.float32),
        pltpu.VMEM((CHUNK, D), jnp.float32),
        pltpu.VMEM_SHARED((N_PAD, D), jnp.float32),
        pltpu.SemaphoreType.DMA,
        pltpu.SemaphoreType.DMA,
    ],
)(_agg_body)

_cnt = functools.partial(
    pl.kernel,
    out_type=jax.ShapeDtypeStruct((NC * N_PAD, D), jnp.float32),
    mesh=_mesh,
    scratch_types=[
        pltpu.VMEM((CHUNK,), jnp.int32),
        pltpu.VMEM((CHUNK, D), jnp.float32),
        pltpu.VMEM_SHARED((N_PAD, D), jnp.float32),
        pltpu.SemaphoreType.DMA,
    ],
)(_cnt_body)


def _mm_t(a, w):
    return lax.dot_general(a, w, (((1,), (1,)), ((), ())),
                           preferred_element_type=jnp.float32)


def _tc_pre_body(x_ref, wl_ref, wr_ref, b_ref, c_ref, yl_ref, yr_ref):
    xv = x_ref[...] + 0.0 * c_ref[...]
    yl_ref[...] = _mm_t(xv, wl_ref[...])
    yr_ref[...] = _mm_t(xv, wr_ref[...]) + b_ref[...]


def _tc_mid_body(p_ref, c_ref, yr_ref, wl_ref, wr_ref, b_ref,
                 yl2_ref, yr2_ref):
    s = p_ref[0] + p_ref[1]
    cnt = jnp.maximum(c_ref[0, :, :1] + c_ref[1, :, :1], 1.0)
    h = jnp.maximum(s / cnt + yr_ref[...], 0.0)
    yl2_ref[...] = _mm_t(h, wl_ref[...])
    yr2_ref[...] = _mm_t(h, wr_ref[...]) + b_ref[...]


def _tc_post_body(p_ref, c_ref, yr_ref, o_ref):
    s = p_ref[0] + p_ref[1]
    cnt = jnp.maximum(c_ref[0, :, :1] + c_ref[1, :, :1], 1.0)
    o_ref[...] = s / cnt + yr_ref[...]


_tc_pre = pl.pallas_call(
    _tc_pre_body,
    out_shape=(jax.ShapeDtypeStruct((N, D), jnp.float32),
               jax.ShapeDtypeStruct((N, D), jnp.float32)),
)

_tc_mid = pl.pallas_call(
    _tc_mid_body,
    out_shape=(jax.ShapeDtypeStruct((N, D), jnp.float32),
               jax.ShapeDtypeStruct((N, D), jnp.float32)),
)

_tc_post = pl.pallas_call(
    _tc_post_body,
    out_shape=jax.ShapeDtypeStruct((N, D), jnp.float32),
)


def kernel(x, edge_index, W1_l, W1_r, b1, W2_l, W2_r, b2):
    src = edge_index[0].astype(jnp.int32)
    dst = edge_index[1].astype(jnp.int32)
    pad = E_PAD - E
    src = jnp.concatenate([src, jnp.zeros((pad,), jnp.int32)])
    dst = jnp.concatenate([dst, jnp.full((pad,), N, jnp.int32)])
    edges_pk = jnp.stack(
        [src.reshape(NW * K_CHUNKS, CHUNK), dst.reshape(NW * K_CHUNKS, CHUNK)],
        axis=1)

    b1r = b1.reshape(1, D)
    b2r = b2.reshape(1, D)

    cnts = _cnt(dst)
    y1l, y1r = _tc_pre(x, W1_l, W1_r, b1r, cnts[:1])
    p1 = _agg(y1l, edges_pk)

    p1 = p1.reshape(NC, N_PAD, D)[:, :N]
    cnt = cnts.reshape(NC, N_PAD, D)[:, :N]

    y2l, y2r = _tc_mid(p1, cnt, y1r, W2_l, W2_r, b2r)
    p2 = _agg(y2l, edges_pk)
    p2 = p2.reshape(NC, N_PAD, D)[:, :N]

    return _tc_post(p2, cnt, y2r)


# speedup vs baseline: 3.6222x; 1.0372x over previous
import functools

import jax
import jax.numpy as jnp
from jax import lax
from jax.experimental import pallas as pl
from jax.experimental.pallas import tpu as pltpu
from jax.experimental.pallas import tpu_sc as plsc

N = 10000
D = 128
E = 320000

NC = 2
NS = 16
NW = NC * NS

CHUNK = 128
N_PAD = 10240
Z = N_PAD // NS
N_PIECES = Z // CHUNK
EDGES_PER_TILE = 10240
E_PAD = EDGES_PER_TILE * NW
K_CHUNKS = EDGES_PER_TILE // CHUNK

_mesh = plsc.VectorSubcoreMesh(core_axis_name="c", subcore_axis_name="s")


def _fill_rows(ref, value):
    vec = jnp.full((16,), value, jnp.float32)

    def body(i, carry):
        for j in range(D // 16):
            ref[i, pl.ds(j * 16, 16)] = vec
        return carry

    lax.fori_loop(0, CHUNK, body, 0)


def _zero_acc(rows, acc, zoff):
    _fill_rows(rows, 0.0)
    for p in range(N_PIECES):
        pltpu.sync_copy(rows, acc.at[pl.ds(zoff + p * CHUNK, CHUNK)])


def _flush_acc(acc, rows, out_hbm, zoff, orow):
    for p in range(N_PIECES):
        pltpu.sync_copy(acc.at[pl.ds(zoff + p * CHUNK, CHUNK)], rows)
        pltpu.sync_copy(rows, out_hbm.at[pl.ds(orow + p * CHUNK, CHUNK)])


def _agg_body(y_hbm, edges_hbm, out_hbm, ib0, ib1, rows0, rows1,
              acc, sem0, sem1):
    cid = lax.axis_index("c")
    sid = lax.axis_index("s")
    wid = sid * NC + cid
    bc = wid * K_CHUNKS
    zoff = sid * Z

    _zero_acc(rows0, acc, zoff)
    plsc.subcore_barrier()

    def load_idx(c, ib):
        pltpu.sync_copy(edges_hbm.at[bc + c], ib)

    def start_gather(ib, rows, sem):
        pltpu.async_copy(y_hbm.at[ib.at[0]], rows, sem)

    def wait_gather(ib, rows, sem):
        pltpu.make_async_copy(y_hbm.at[ib.at[0]], rows, sem).wait()

    def scatter(rows, ib):
        pltpu.sync_copy(rows, acc.at[ib.at[1]], add=True)

    load_idx(0, ib0)
    start_gather(ib0, rows0, sem0)
    G = K_CHUNKS // 2

    def pair(g, carry):
        c = 2 * g
        load_idx(c + 1, ib1)
        start_gather(ib1, rows1, sem1)
        wait_gather(ib0, rows0, sem0)
        scatter(rows0, ib0)

        @pl.when(g + 1 < G)
        def _():
            load_idx(c + 2, ib0)
            start_gather(ib0, rows0, sem0)

        wait_gather(ib1, rows1, sem1)
        scatter(rows1, ib1)
        return carry

    lax.fori_loop(0, G, pair, 0)
    plsc.subcore_barrier()
    _flush_acc(acc, rows0, out_hbm, zoff, cid * N_PAD + zoff)


def _cnt_body(edges_hbm, out_hbm, ib0, ib1, rows, acc, sem):
    cid = lax.axis_index("c")
    sid = lax.axis_index("s")
    wid = sid * NC + cid
    bc = wid * K_CHUNKS
    zoff = sid * Z

    _zero_acc(rows, acc, zoff)
    _fill_rows(rows, 1.0)
    plsc.subcore_barrier()

    def load_idx(c, ib):
        pltpu.sync_copy(edges_hbm.at[bc + c], ib)

    def scatter(ib):
        pltpu.sync_copy(rows, acc.at[ib.at[1]], add=True)

    load_idx(0, ib0)
    G = K_CHUNKS // 2

    def pair(g, carry):
        c = 2 * g
        load_idx(c + 1, ib1)
        scatter(ib0)

        @pl.when(g + 1 < G)
        def _():
            load_idx(c + 2, ib0)

        scatter(ib1)
        return carry

    lax.fori_loop(0, G, pair, 0)
    plsc.subcore_barrier()
    _flush_acc(acc, rows, out_hbm, zoff, cid * N_PAD + zoff)


_agg = functools.partial(
    pl.kernel,
    out_type=jax.ShapeDtypeStruct((NC * N_PAD, D), jnp.float32),
    mesh=_mesh,
    scratch_types=[
        pltpu.VMEM((2, CHUNK), jnp.int32),
        pltpu.VMEM((2, CHUNK), jnp.int32),
        pltpu.VMEM((CHUNK, D), jnp.float32),
        pltpu.VMEM((CHUNK, D), jnp.float32),
        pltpu.VMEM_SHARED((N_PAD, D), jnp.float32),
        pltpu.SemaphoreType.DMA,
        pltpu.SemaphoreType.DMA,
    ],
)(_agg_body)

_cnt = functools.partial(
    pl.kernel,
    out_type=jax.ShapeDtypeStruct((NC * N_PAD, D), jnp.float32),
    mesh=_mesh,
    scratch_types=[
        pltpu.VMEM((2, CHUNK), jnp.int32),
        pltpu.VMEM((2, CHUNK), jnp.int32),
        pltpu.VMEM((CHUNK, D), jnp.float32),
        pltpu.VMEM_SHARED((N_PAD, D), jnp.float32),
        pltpu.SemaphoreType.DMA,
    ],
)(_cnt_body)


def _mm_t(a, w):
    return lax.dot_general(a, w, (((1,), (1,)), ((), ())),
                           preferred_element_type=jnp.float32)


def _tc_pre_body(x_ref, wl_ref, wr_ref, b_ref, c_ref, yl_ref, yr_ref):
    xv = x_ref[...] + 0.0 * c_ref[...]
    yl_ref[...] = _mm_t(xv, wl_ref[...])
    yr_ref[...] = _mm_t(xv, wr_ref[...]) + b_ref[...]


def _tc_mid_body(p_ref, c_ref, yr_ref, wl_ref, wr_ref, b_ref,
                 yl2_ref, yr2_ref):
    s = p_ref[0] + p_ref[1]
    cnt = jnp.maximum(c_ref[0, :, :1] + c_ref[1, :, :1], 1.0)
    h = jnp.maximum(s / cnt + yr_ref[...], 0.0)
    yl2_ref[...] = _mm_t(h, wl_ref[...])
    yr2_ref[...] = _mm_t(h, wr_ref[...]) + b_ref[...]


def _tc_post_body(p_ref, c_ref, yr_ref, o_ref):
    s = p_ref[0] + p_ref[1]
    cnt = jnp.maximum(c_ref[0, :, :1] + c_ref[1, :, :1], 1.0)
    o_ref[...] = s / cnt + yr_ref[...]


_tc_pre = pl.pallas_call(
    _tc_pre_body,
    out_shape=(jax.ShapeDtypeStruct((N, D), jnp.float32),
               jax.ShapeDtypeStruct((N, D), jnp.float32)),
)

_tc_mid = pl.pallas_call(
    _tc_mid_body,
    out_shape=(jax.ShapeDtypeStruct((N, D), jnp.float32),
               jax.ShapeDtypeStruct((N, D), jnp.float32)),
)

_tc_post = pl.pallas_call(
    _tc_post_body,
    out_shape=jax.ShapeDtypeStruct((N, D), jnp.float32),
)


def kernel(x, edge_index, W1_l, W1_r, b1, W2_l, W2_r, b2):
    src = edge_index[0].astype(jnp.int32)
    dst = edge_index[1].astype(jnp.int32)
    pad = E_PAD - E
    src = jnp.concatenate([src, jnp.zeros((pad,), jnp.int32)])
    dst = jnp.concatenate([dst, jnp.full((pad,), N, jnp.int32)])
    edges_pk = jnp.stack(
        [src.reshape(NW * K_CHUNKS, CHUNK), dst.reshape(NW * K_CHUNKS, CHUNK)],
        axis=1)

    b1r = b1.reshape(1, D)
    b2r = b2.reshape(1, D)

    cnts = _cnt(edges_pk)
    y1l, y1r = _tc_pre(x, W1_l, W1_r, b1r, cnts[:1])
    p1 = _agg(y1l, edges_pk)

    p1 = p1.reshape(NC, N_PAD, D)[:, :N]
    cnt = cnts.reshape(NC, N_PAD, D)[:, :N]

    y2l, y2r = _tc_mid(p1, cnt, y1r, W2_l, W2_r, b2r)
    p2 = _agg(y2l, edges_pk)
    p2 = p2.reshape(NC, N_PAD, D)[:, :N]

    return _tc_post(p2, cnt, y2r)
